# Initial kernel scaffold; baseline (speedup 1.0000x reference)
import jax
import jax.numpy as jnp
from jax.experimental import pallas as pl


def kernel(x, edge_index, edge_weight, batch, pool, eps_0, W1_0, b1_0, W2_0, b2_0, gamma_0, beta_0, eps_1, W1_1, b1_1, W2_1, b2_1, gamma_1, beta_1, eps_2, W1_2, b1_2, W2_2, b2_2, gamma_2, beta_2):
    raise NotImplementedError("write your pallas kernel here")


# speedup vs baseline: 2.6172x; 2.6172x over previous
import functools

import jax
import jax.numpy as jnp
from jax import lax
from jax.experimental import pallas as pl
from jax.experimental.pallas import tpu as pltpu
from jax.experimental.pallas import tpu_sc as plsc

N = 10000
E = 320000
D = 128
G = 128

NC = 2
NS = 16
NW = NC * NS

CHUNK = 128
CH_PT = 80
NCHUNKS = CH_PT * NW
EPAD = NCHUNKS * CHUNK - E
ROWS_PT = 624

BR = 2000
NBLK = N // BR


def _sc_agg_body(h_hbm, src_hbm, dst_hbm, w_hbm, zero_hbm, out_hbm,
                 src_v, dst_v, w_v, rows, acc, gsem):
    c = lax.axis_index("c")
    s = lax.axis_index("s")
    wid = s * NC + c

    c0 = wid * CH_PT
    pltpu.sync_copy(src_hbm.at[pl.ds(c0, CH_PT)], src_v)
    pltpu.sync_copy(dst_hbm.at[pl.ds(c0, CH_PT)], dst_v)
    pltpu.sync_copy(w_hbm.at[pl.ds(c0 * CHUNK, CH_PT * CHUNK)], w_v)

    r0 = s * ROWS_PT
    rtail = NS * ROWS_PT
    pltpu.sync_copy(zero_hbm.at[pl.ds(r0, ROWS_PT)], acc.at[pl.ds(r0, ROWS_PT)])

    @pl.when(s == NS - 1)
    def _():
        pltpu.sync_copy(zero_hbm.at[pl.ds(rtail, N - rtail)],
                        acc.at[pl.ds(rtail, N - rtail)])

    plsc.subcore_barrier()

    def step(g, carry):
        pltpu.async_copy(h_hbm.at[src_v.at[g]], rows, gsem).wait()

        def srow(j, carry2):
            wsp = plsc.load_gather(w_v, [jnp.full((16,), g * CHUNK, jnp.int32)
                                         + j])
            for t in range(D // 16):
                rows[j, pl.ds(t * 16, 16)] = rows[j, pl.ds(t * 16, 16)] * wsp
            return carry2

        lax.fori_loop(0, CHUNK, srow, 0, unroll=False)

        pltpu.sync_copy(rows, acc.at[dst_v.at[g]], add=True)
        return carry

    lax.fori_loop(0, CH_PT, step, 0, unroll=False)
    plsc.subcore_barrier()

    pltpu.sync_copy(acc.at[pl.ds(r0, ROWS_PT)], out_hbm.at[c, pl.ds(r0, ROWS_PT)])

    @pl.when(s == NS - 1)
    def _():
        pltpu.sync_copy(acc.at[pl.ds(rtail, N - rtail)],
                        out_hbm.at[c, pl.ds(rtail, N - rtail)])


_sc_agg = functools.partial(
    pl.kernel,
    out_type=jax.ShapeDtypeStruct((NC, N, D), jnp.float32),
    mesh=plsc.VectorSubcoreMesh(core_axis_name="c", subcore_axis_name="s"),
    scratch_types=[
        pltpu.VMEM((CH_PT, CHUNK), jnp.int32),
        pltpu.VMEM((CH_PT, CHUNK), jnp.int32),
        pltpu.VMEM((CH_PT * CHUNK,), jnp.float32),
        pltpu.VMEM((CHUNK, D), jnp.float32),
        pltpu.MemorySpace.VMEM_SHARED((N, D), jnp.float32),
        pltpu.SemaphoreType.DMA,
    ],
    compiler_params=pltpu.CompilerParams(needs_layout_passes=False),
)(_sc_agg_body)


def _mlp_body(h_ref, p0_ref, p1_ref, w1_ref, w2_ref, vec_ref, r_ref, st_ref):
    i = pl.program_id(0)
    eps_row = vec_ref[4, :]
    z = h_ref[...] * (1.0 + eps_row)[None, :] + p0_ref[...] + p1_ref[...]
    a = jnp.maximum(
        lax.dot(z, w1_ref[...], preferred_element_type=jnp.float32)
        + vec_ref[0, :][None, :], 0.0)
    u = (lax.dot(a, w2_ref[...], preferred_element_type=jnp.float32)
         + vec_ref[1, :][None, :])
    r = jnp.maximum(u, 0.0)
    r_ref[...] = r
    ps = jnp.concatenate(
        [jnp.sum(r, 0, keepdims=True),
         jnp.sum(r * r, 0, keepdims=True),
         jnp.zeros((6, D), jnp.float32)], 0)

    @pl.when(i == 0)
    def _():
        st_ref[...] = ps

    @pl.when(i > 0)
    def _():
        st_ref[...] += ps


def _mlp_call(h, p0, p1, w1, w2, vecs):
    return pl.pallas_call(
        _mlp_body,
        grid=(NBLK,),
        in_specs=[
            pl.BlockSpec((BR, D), lambda i: (i, 0)),
            pl.BlockSpec((BR, D), lambda i: (i, 0)),
            pl.BlockSpec((BR, D), lambda i: (i, 0)),
            pl.BlockSpec((D, D), lambda i: (0, 0)),
            pl.BlockSpec((D, D), lambda i: (0, 0)),
            pl.BlockSpec((8, D), lambda i: (0, 0)),
        ],
        out_specs=[
            pl.BlockSpec((BR, D), lambda i: (i, 0)),
            pl.BlockSpec((8, D), lambda i: (0, 0)),
        ],
        out_shape=[
            jax.ShapeDtypeStruct((N, D), jnp.float32),
            jax.ShapeDtypeStruct((8, D), jnp.float32),
        ],
        compiler_params=pltpu.CompilerParams(
            dimension_semantics=("arbitrary",)),
    )(h, p0, p1, w1, w2, vecs)


def _bn_body(r_ref, st_ref, vec_ref, b_ref, hn_ref, pg_ref):
    i = pl.program_id(0)
    m = st_ref[0, :] * (1.0 / N)
    v = st_ref[1, :] * (1.0 / N) - m * m
    inv = lax.rsqrt(v + 1e-5)
    gamma = vec_ref[2, :]
    beta = vec_ref[3, :]
    scale = gamma * inv
    shift = beta - m * scale
    hn = r_ref[...] * scale[None, :] + shift[None, :]
    hn_ref[...] = hn
    b = b_ref[0, 0, :]
    gi = lax.broadcasted_iota(jnp.int32, (G, BR), 0)
    oh = (gi == b[None, :]).astype(jnp.float32)
    pg = lax.dot(oh, hn, preferred_element_type=jnp.float32)

    @pl.when(i == 0)
    def _():
        pg_ref[...] = pg

    @pl.when(i > 0)
    def _():
        pg_ref[...] += pg


def _bn_call(r, st, vecs, batch3):
    return pl.pallas_call(
        _bn_body,
        grid=(NBLK,),
        in_specs=[
            pl.BlockSpec((BR, D), lambda i: (i, 0)),
            pl.BlockSpec((8, D), lambda i: (0, 0)),
            pl.BlockSpec((8, D), lambda i: (0, 0)),
            pl.BlockSpec((1, 1, BR), lambda i: (i, 0, 0)),
        ],
        out_specs=[
            pl.BlockSpec((BR, D), lambda i: (i, 0)),
            pl.BlockSpec((G, D), lambda i: (0, 0)),
        ],
        out_shape=[
            jax.ShapeDtypeStruct((N, D), jnp.float32),
            jax.ShapeDtypeStruct((G, D), jnp.float32),
        ],
        compiler_params=pltpu.CompilerParams(
            dimension_semantics=("arbitrary",)),
    )(r, st, vecs, batch3)


def kernel(x, edge_index, edge_weight, batch, pool,
           eps_0, W1_0, b1_0, W2_0, b2_0, gamma_0, beta_0,
           eps_1, W1_1, b1_1, W2_1, b2_1, gamma_1, beta_1,
           eps_2, W1_2, b1_2, W2_2, b2_2, gamma_2, beta_2):
    pad_i = jnp.zeros((EPAD,), jnp.int32)
    src2 = jnp.concatenate([edge_index[0], pad_i]).reshape(NCHUNKS, CHUNK)
    dst2 = jnp.concatenate([edge_index[1], pad_i]).reshape(NCHUNKS, CHUNK)
    w1d = jnp.concatenate([edge_weight, jnp.zeros((EPAD,), jnp.float32)])
    zeros = jnp.zeros((N, D), jnp.float32)
    batch3 = batch.reshape(NBLK, 1, BR)

    layers = [
        (eps_0, W1_0, b1_0, W2_0, b2_0, gamma_0, beta_0),
        (eps_1, W1_1, b1_1, W2_1, b2_1, gamma_1, beta_1),
        (eps_2, W1_2, b1_2, W2_2, b2_2, gamma_2, beta_2),
    ]

    h = x
    hs = []
    pooled = []
    for (eps, w1, b1, w2, b2, gamma, beta) in layers:
        vecs = jnp.concatenate(
            [b1[None, :], b2[None, :], gamma[None, :], beta[None, :],
             jnp.full((1, D), eps, jnp.float32),
             jnp.zeros((3, D), jnp.float32)], 0)
        parts = _sc_agg(h, src2, dst2, w1d, zeros)
        r, st = _mlp_call(h, parts[0], parts[1], w1, w2, vecs)
        hn, pg = _bn_call(r, st, vecs, batch3)
        h = hn
        hs.append(hn)
        pooled.append(pg)

    return jnp.concatenate(pooled, axis=1), jnp.concatenate(hs, axis=1)
